# baseline (speedup 1.0000x reference)
import functools

import jax
import jax.numpy as jnp
from jax import lax
from jax.experimental import pallas as pl
from jax.experimental.pallas import tpu as pltpu
from jax.experimental.pallas import tpu_sc as plsc


@functools.lru_cache(maxsize=None)
def _build(B, D, V_u, V_i):
    info = plsc.get_sparse_core_info()
    NC, NS, L = info.num_cores, info.num_subcores, info.num_lanes
    NW = NC * NS
    BPW = B // NW
    CHUNK = 128
    NCHUNK = BPW // CHUNK

    mesh = plsc.VectorSubcoreMesh(core_axis_name="c", subcore_axis_name="s")

    @functools.partial(
        pl.kernel,
        mesh=mesh,
        compiler_params=pltpu.CompilerParams(
            needs_layout_passes=False, use_tc_tiling_on_sc=False
        ),
        out_type=(
            jax.ShapeDtypeStruct((NW, BPW), jnp.float32),
            jax.ShapeDtypeStruct((NW, BPW), jnp.float32),
        ),
        scratch_types=[
            pltpu.VMEM((NCHUNK, CHUNK), jnp.int32),
            pltpu.VMEM((NCHUNK, CHUNK), jnp.int32),
            pltpu.VMEM((NCHUNK, CHUNK), jnp.int32),
            pltpu.VMEM((BPW, D), jnp.float32),
            pltpu.VMEM((BPW, D), jnp.float32),
            pltpu.VMEM((BPW, D), jnp.float32),
            pltpu.VMEM((BPW,), jnp.float32),
            pltpu.VMEM((BPW,), jnp.float32),
            pltpu.SemaphoreType.DMA,
        ],
    )
    def bpr(u_hbm, i_hbm, j_hbm, ut_hbm, it_hbm, pos_hbm, neg_hbm,
            uidx, iidx, jidx, urows, irows, jrows, posv, negv, sem):
        wid = lax.axis_index("s") * NC + lax.axis_index("c")
        pltpu.sync_copy(u_hbm.at[wid], uidx)
        pltpu.sync_copy(i_hbm.at[wid], iidx)
        pltpu.sync_copy(j_hbm.at[wid], jidx)
        copies = []
        for c in range(NCHUNK):
            dst = pl.ds(c * CHUNK, CHUNK)
            copies.append(pltpu.async_copy(ut_hbm.at[uidx.at[c]], urows.at[dst], sem))
            copies.append(pltpu.async_copy(it_hbm.at[iidx.at[c]], irows.at[dst], sem))
            copies.append(pltpu.async_copy(it_hbm.at[jidx.at[c]], jrows.at[dst], sem))
        for cp in copies:
            cp.wait()

        lanes = lax.iota(jnp.int32, L)

        def block(bb, carry):
            rows = lanes + bb * L
            accp = jnp.zeros((L,), jnp.float32)
            accn = jnp.zeros((L,), jnp.float32)
            for d in range(D):
                cols = jnp.full((L,), d, jnp.int32)
                gu = plsc.load_gather(urows, [rows, cols])
                gi = plsc.load_gather(irows, [rows, cols])
                gj = plsc.load_gather(jrows, [rows, cols])
                accp = accp + gu * gi
                accn = accn + gu * gj
            posv[pl.ds(bb * L, L)] = accp
            negv[pl.ds(bb * L, L)] = accn
            return carry

        lax.fori_loop(0, BPW // L, block, 0)
        pltpu.sync_copy(posv, pos_hbm.at[wid])
        pltpu.sync_copy(negv, neg_hbm.at[wid])

    def run(u, i, j, user_table, item_table):
        pos, neg = bpr(
            u.reshape(NW, NCHUNK, CHUNK),
            i.reshape(NW, NCHUNK, CHUNK),
            j.reshape(NW, NCHUNK, CHUNK),
            user_table,
            item_table,
        )
        return pos.reshape(B), neg.reshape(B)

    return run


def kernel(u, i, j, user_table, item_table):
    B = u.shape[0]
    D = user_table.shape[1]
    run = _build(B, D, user_table.shape[0], item_table.shape[0])
    return run(u, i, j, user_table, item_table)
